# baseline (speedup 1.0000x reference)
import functools

import jax
import jax.numpy as jnp
from jax import lax
from jax.experimental import pallas as pl
from jax.experimental.pallas import tpu as pltpu
from jax.experimental.pallas import tpu_sc as plsc

N = 10000
D = 128
E = 320000

NC = 2
NS = 16
NW = NC * NS

ACC_ROWS = 10040
EDGE_BLK = 128
EDGE_BLOCKS = 2688
BLKS_PER_TILE = EDGE_BLOCKS // NW
E_PAD = EDGE_BLOCKS * EDGE_BLK
ROWS_PER_TILE = 640
NBUF = 3
CHUNK = 3
N_ITERS = BLKS_PER_TILE // (2 * CHUNK)


def _sc_aggregate(h, ids3d):
  mesh = plsc.VectorSubcoreMesh(
      core_axis_name="c", subcore_axis_name="s", num_cores=NC, num_subcores=NS)

  @functools.partial(
      pl.kernel,
      out_type=jax.ShapeDtypeStruct((NC, N, D), jnp.float32),
      mesh=mesh,
      scratch_types=[
          pltpu.VMEM((2, CHUNK, 2, EDGE_BLK), jnp.int32),
          pltpu.VMEM((NBUF, EDGE_BLK, D), jnp.float32),
          pltpu.VMEM_SHARED((ACC_ROWS, D), jnp.float32),
          pltpu.SemaphoreType.DMA,
          pltpu.SemaphoreType.DMA,
          pltpu.SemaphoreType.DMA,
      ] + [pltpu.SemaphoreType.DMA] * NBUF,
  )
  def agg_kernel(h_hbm, ids_hbm, out_hbm,
                 idsv, rows, acc, isem, idsem0, idsem1, *gsems):
    idsems = (idsem0, idsem1)
    c = lax.axis_index("c")
    s = lax.axis_index("s")
    wid = s * NC + c

    r0 = s * ROWS_PER_TILE
    blk0 = wid * BLKS_PER_TILE

    def load_ids(chunk, slot, sem):
      pltpu.async_copy(ids_hbm.at[pl.ds(blk0 + chunk * CHUNK, CHUNK)],
                       idsv.at[slot], sem)

    def wait_ids(slot, sem):
      pltpu.make_async_copy(ids_hbm.at[pl.ds(blk0, CHUNK)],
                            idsv.at[slot], sem).wait()

    def gather(j, slot, b):
      pltpu.async_copy(h_hbm.at[idsv.at[slot, b, 0]], rows.at[b], gsems[b])

    def wait_gather(b):
      pltpu.make_async_copy(h_hbm.at[idsv.at[0, 0, 0]], rows.at[b],
                            gsems[b]).wait()

    def scatter(slot, b):
      pltpu.sync_copy(rows.at[b], acc.at[idsv.at[slot, b, 1]], add=True)

    init_spans_main = tuple((k * 128, 128) for k in range(5))
    init_spans_tail = ((0, 128), (128, 128), (256, 128), (384, 8), (392, 8))

    @pl.when(s < NS - 1)
    def _():
      for off, sz in init_spans_main:
        pltpu.async_copy(h_hbm.at[pl.ds(r0 + off, sz)],
                         acc.at[pl.ds(r0 + off, sz)], isem)

    @pl.when(s == NS - 1)
    def _():
      tail0 = (NS - 1) * ROWS_PER_TILE
      for off, sz in init_spans_tail:
        pltpu.async_copy(h_hbm.at[pl.ds(tail0 + off, sz)],
                         acc.at[pl.ds(tail0 + off, sz)], isem)

    load_ids(0, 0, idsems[0])
    load_ids(1, 1, idsems[1])
    wait_ids(0, idsems[0])
    for b in range(NBUF):
      gather(b, 0, b)

    @pl.when(s < NS - 1)
    def _():
      for off, sz in init_spans_main:
        pltpu.make_async_copy(h_hbm.at[pl.ds(r0 + off, sz)],
                              acc.at[pl.ds(r0 + off, sz)], isem).wait()

    @pl.when(s == NS - 1)
    def _():
      tail0 = (NS - 1) * ROWS_PER_TILE
      for off, sz in init_spans_tail:
        pltpu.make_async_copy(h_hbm.at[pl.ds(tail0 + off, sz)],
                              acc.at[pl.ds(tail0 + off, sz)], isem).wait()

    plsc.subcore_barrier()

    def step(g, carry):
      wait_ids(1, idsems[1])
      for b in range(NBUF):
        wait_gather(b)
        scatter(0, b)
        gather(b, 1, b)

      @pl.when(g < N_ITERS - 1)
      def _():
        load_ids(2 * g + 2, 0, idsems[0])

      for b in range(NBUF):
        wait_gather(b)
        scatter(1, b)

        if b == 0:
          @pl.when(g < N_ITERS - 1)
          def _():
            wait_ids(0, idsems[0])

        @pl.when(g < N_ITERS - 1)
        def _():
          gather(b, 0, b)

      @pl.when(g < N_ITERS - 1)
      def _():
        load_ids(2 * g + 3, 1, idsems[1])

      return carry

    lax.fori_loop(0, N_ITERS, step, 0)

    plsc.subcore_barrier()

    @pl.when(s < NS - 1)
    def _():
      pltpu.sync_copy(acc.at[pl.ds(r0, ROWS_PER_TILE)],
                      out_hbm.at[c, pl.ds(r0, ROWS_PER_TILE)])

    @pl.when(s == NS - 1)
    def _():
      tail0 = (NS - 1) * ROWS_PER_TILE
      pltpu.sync_copy(acc.at[pl.ds(tail0, N - tail0)],
                      out_hbm.at[c, pl.ds(tail0, N - tail0)])

  return agg_kernel(h, ids3d)


ROW_BLK = 2000


def _mlp1_body(p_ref, h_ref, wa_ref, ba_ref, wb_ref, bb_ref, o_ref):
  z = p_ref[0] + p_ref[1] - h_ref[...]
  h = jnp.maximum(
      jnp.dot(z, wa_ref[...], preferred_element_type=jnp.float32)
      + ba_ref[...], 0.0)
  o_ref[...] = (
      jnp.dot(h, wb_ref[...], preferred_element_type=jnp.float32)
      + bb_ref[...])


def _mlp2_body(p_ref, h_ref, wa_ref, ba_ref, wb_ref, bb_ref, wfc_ref,
               bfc_ref, o_ref, acc_ref):
  i = pl.program_id(0)
  z = p_ref[0] + p_ref[1] - h_ref[...]
  h = jnp.maximum(
      jnp.dot(z, wa_ref[...], preferred_element_type=jnp.float32)
      + ba_ref[...], 0.0)
  h = (jnp.dot(h, wb_ref[...], preferred_element_type=jnp.float32)
       + bb_ref[...])
  psum = jnp.sum(h, axis=0, keepdims=True)

  @pl.when(i == 0)
  def _():
    acc_ref[...] = psum

  @pl.when(i > 0)
  def _():
    acc_ref[...] = acc_ref[...] + psum

  @pl.when(i == (N // ROW_BLK) - 1)
  def _():
    pooled = acc_ref[...] * (1.0 / N)
    o_ref[...] = (
        jnp.dot(pooled, wfc_ref[...], preferred_element_type=jnp.float32)
        + bfc_ref[...])


def _full_spec(shape):
  return pl.BlockSpec(shape, lambda i: tuple(0 for _ in shape))


def kernel(x, edge_index, W1a, b1a, W1b, b1b, W2a, b2a, W2b, b2b, Wfc, bfc):
  src = edge_index[0].astype(jnp.int32)
  dst = edge_index[1].astype(jnp.int32)
  pad_e = E_PAD - E
  pad_iota = jnp.arange(pad_e, dtype=jnp.int32)
  src2d = jnp.concatenate([src, pad_iota % N]).reshape(EDGE_BLOCKS, EDGE_BLK)
  dst2d = jnp.concatenate([dst, N + pad_iota % (ACC_ROWS - N)]).reshape(
      EDGE_BLOCKS, EDGE_BLK)
  ids3d = jnp.stack([src2d, dst2d], axis=1)

  b1a2, b1b2 = b1a.reshape(1, D), b1b.reshape(1, D)
  b2a2, b2b2 = b2a.reshape(1, D), b2b.reshape(1, D)
  bfc2 = bfc.reshape(1, D)

  grid = (N // ROW_BLK,)
  p_spec = pl.BlockSpec((NC, ROW_BLK, D), lambda i: (0, i, 0))
  h_spec = pl.BlockSpec((ROW_BLK, D), lambda i: (i, 0))
  w_spec = _full_spec((D, D))
  b_spec = _full_spec((1, D))

  p1 = _sc_aggregate(x, ids3d)

  h1 = pl.pallas_call(
      _mlp1_body,
      grid=grid,
      in_specs=[p_spec, h_spec, w_spec, b_spec, w_spec, b_spec],
      out_specs=pl.BlockSpec((ROW_BLK, D), lambda i: (i, 0)),
      out_shape=jax.ShapeDtypeStruct((N, D), jnp.float32),
  )(p1, x, W1a, b1a2, W1b, b1b2)

  p2 = _sc_aggregate(h1, ids3d)

  out = pl.pallas_call(
      _mlp2_body,
      grid=grid,
      in_specs=[p_spec, h_spec, w_spec, b_spec, w_spec, b_spec, w_spec,
                b_spec],
      out_specs=pl.BlockSpec((1, D), lambda i: (0, 0)),
      out_shape=jax.ShapeDtypeStruct((1, D), jnp.float32),
      scratch_shapes=[pltpu.VMEM((1, D), jnp.float32)],
  )(p2, h1, W2a, b2a2, W2b, b2b2, Wfc, bfc2)

  return out[0]
